# baseline (speedup 1.0000x reference)
import functools

import jax
import jax.numpy as jnp
from jax import lax
from jax.experimental import pallas as pl
from jax.experimental.pallas import tpu as pltpu
from jax.experimental.pallas import tpu_sc as plsc

NC = 2
NS = 16
NW = NC * NS
L = 16

B = 16384
E = 64
BPW = B // NW
NCHUNK = 4
CHUNK = BPW // NCHUNK
IDX_ROWS_PER_W = BPW // 128


def _sc_body(uidx_hbm, cidx_hbm, utab_hbm, ctab_hbm, w_hbm, b_hbm, out_hbm,
             uidx_v, cidx_v, urows_v, crows_v, w_v, b_v, out_v, sem):
    c = lax.axis_index("c")
    s = lax.axis_index("s")
    wid = s * NC + c

    base_row = wid * IDX_ROWS_PER_W
    pltpu.sync_copy(uidx_hbm.at[pl.ds(base_row, IDX_ROWS_PER_W)], uidx_v)
    pltpu.sync_copy(cidx_hbm.at[pl.ds(base_row, IDX_ROWS_PER_W)], cidx_v)
    pltpu.sync_copy(w_hbm, w_v)
    pltpu.sync_copy(b_hbm, b_v)

    copies = []
    for j in range(NCHUNK):
        copies.append(pltpu.async_copy(
            utab_hbm.at[uidx_v.at[j]], urows_v.at[pl.ds(j * CHUNK, CHUNK)],
            sem))
        copies.append(pltpu.async_copy(
            ctab_hbm.at[cidx_v.at[j]], crows_v.at[pl.ds(j * CHUNK, CHUNK)],
            sem))
    for cp in copies:
        cp.wait()

    wv = w_v[...]
    bv = b_v[...]
    iota = lax.iota(jnp.int32, L)

    def blk_body(i, carry):
        row = i * L + iota
        acc = jnp.zeros((L,), jnp.float32)
        for d in range(E):
            col = jnp.full((L,), d, jnp.int32)
            gu = plsc.load_gather(urows_v, [row, col])
            gc = plsc.load_gather(crows_v, [row, col])
            acc = acc + gu * gc
        x = acc * wv + bv
        y = 1.0 / (1.0 + jnp.exp(-x))
        out_v[pl.ds(i * L, L)] = y
        return carry

    lax.fori_loop(0, BPW // L, blk_body, 0)
    pltpu.sync_copy(out_v, out_hbm.at[pl.ds(wid * BPW, BPW)])


@jax.jit
def _run(uidx, cidx, user_table, content_table, wvec, bvec):
    mesh = plsc.VectorSubcoreMesh(
        core_axis_name="c", subcore_axis_name="s",
        num_cores=NC, num_subcores=NS)
    return pl.kernel(
        _sc_body,
        out_type=jax.ShapeDtypeStruct((B,), jnp.float32),
        mesh=mesh,
        compiler_params=pltpu.CompilerParams(
            needs_layout_passes=False, use_tc_tiling_on_sc=False),
        scratch_types=[
            pltpu.VMEM((IDX_ROWS_PER_W, 128), jnp.int32),
            pltpu.VMEM((IDX_ROWS_PER_W, 128), jnp.int32),
            pltpu.VMEM((BPW, E), jnp.float32),
            pltpu.VMEM((BPW, E), jnp.float32),
            pltpu.VMEM((L,), jnp.float32),
            pltpu.VMEM((L,), jnp.float32),
            pltpu.VMEM((BPW,), jnp.float32),
            pltpu.SemaphoreType.DMA,
        ],
    )(uidx, cidx, user_table, content_table, wvec, bvec)


def kernel(inputs, user_table, content_table, dense_w, dense_b):
    uidx = inputs[:, 0].reshape(128, 128)
    cidx = inputs[:, 1].reshape(128, 128)
    wvec = jnp.full((L,), dense_w[0, 0], jnp.float32)
    bvec = jnp.full((L,), dense_b[0], jnp.float32)
    out = _run(uidx, cidx, user_table, content_table, wvec, bvec)
    return out.reshape(B, 1)
